# baseline (speedup 1.0000x reference)
import functools
import math

import jax
import jax.numpy as jnp
from jax.experimental import pallas as pl
from jax.experimental.pallas import tpu as pltpu


def _stats(agg, sum_ref, sq_ref):
    sum_ref[...] = jnp.sum(agg, axis=0, keepdims=True)[None]
    sq_ref[...] = jnp.sum(agg * agg, axis=0, keepdims=True)[None]


def _layer1_kernel(x_ref, w_ref, s_ref, agg_ref, sum_ref, sq_ref, *, groups, gin):
    parts = []
    for i in range(groups):
        xg = x_ref[pl.ds(i * gin, gin), :]
        parts.append(jax.lax.dot_general(
            xg, w_ref[...], (((0,), (0,)), ((), ())),
            preferred_element_type=jnp.float32))
    hw = jnp.concatenate(parts, axis=1)
    agg = jnp.dot(s_ref[...], hw, preferred_element_type=jnp.float32)
    agg_ref[...] = agg.astype(agg_ref.dtype)
    _stats(agg, sum_ref, sq_ref)


def _fold_stats(ps_ref, pq_ref, gamma_ref, beta_ref, p_ref, n_nodes, eps):
    tot_jd = jnp.sum(ps_ref[...], axis=0)
    tot2_jd = jnp.sum(pq_ref[...], axis=0)
    dn = (((1,), (0,)), ((), ()))
    tot = jax.lax.dot_general(tot_jd, p_ref[...], dn,
                              preferred_element_type=jnp.float32)
    tot2 = jax.lax.dot_general(tot2_jd, p_ref[...], dn,
                               preferred_element_type=jnp.float32)
    mean = tot / n_nodes
    var = tot2 / n_nodes - mean * mean
    scale_c = gamma_ref[...] * jax.lax.rsqrt(var + eps)
    shift_c = beta_ref[...] - mean * scale_c
    dt = (((1,), (1,)), ((), ()))
    scale = jax.lax.dot_general(scale_c, p_ref[...], dt,
                                preferred_element_type=jnp.float32)
    shift = jax.lax.dot_general(shift_c, p_ref[...], dt,
                                preferred_element_type=jnp.float32)
    return scale, shift


def _layer_kernel(h_ref, ps_ref, pq_ref, gamma_ref, beta_ref, p_ref,
                  w_ref, s_ref, agg_ref, sum_ref, sq_ref, *,
                  groups, gin, n_nodes, eps):
    scale, shift = _fold_stats(ps_ref, pq_ref, gamma_ref, beta_ref, p_ref,
                               n_nodes, eps)
    h = jnp.maximum(h_ref[...].astype(jnp.float32) * scale + shift, 0.0)
    parts = []
    for i in range(groups):
        hg = h[:, i * gin:(i + 1) * gin]
        parts.append(jnp.dot(hg, w_ref[...],
                             preferred_element_type=jnp.float32))
    hw = jnp.concatenate(parts, axis=1)
    agg = jnp.dot(s_ref[...], hw, preferred_element_type=jnp.float32)
    agg_ref[...] = agg.astype(agg_ref.dtype)
    _stats(agg, sum_ref, sq_ref)


def _bn_relu_t_kernel(agg_ref, ps_ref, pq_ref, gamma_ref, beta_ref, p_ref,
                      eye_ref, o_ref, *, groups, gout, bt, cout, n_nodes, eps):
    scale, shift = _fold_stats(ps_ref, pq_ref, gamma_ref, beta_ref, p_ref,
                               n_nodes, eps)
    y = jnp.maximum(agg_ref[...].astype(jnp.float32) * scale + shift, 0.0)
    parts = []
    for i in range(groups):
        yg = y[:, i * gout:(i + 1) * gout]
        parts.append(jax.lax.dot_general(
            yg, eye_ref[...], (((0,), (0,)), ((), ())),
            preferred_element_type=jnp.float32))
    o_ref[...] = jnp.concatenate(parts, axis=0).reshape(bt, cout, -1)


def _normalized_adjacency(edge_index, num_nodes):
    src, dst = edge_index[0], edge_index[1]
    oh_src = jax.nn.one_hot(src, num_nodes, dtype=jnp.float32)
    oh_dst = jax.nn.one_hot(dst, num_nodes, dtype=jnp.float32)
    a = jax.lax.dot_general(oh_dst, oh_src, (((0,), (0,)), ((), ())))
    a = a + 2.0 * jnp.eye(num_nodes, dtype=jnp.float32)
    deg = jnp.sum(a, axis=1)
    dinv = jnp.where(deg > 0, jax.lax.rsqrt(deg), 0.0)
    return dinv[:, None] * a * dinv[None, :]


def kernel(x, edge_index, w1, b1, g1, be1, w2, b2, g2, be2, w3, b3, g3, be3):
    b, n, c0, l = x.shape
    B = b * n
    c1, c2, c3 = w1.shape[1], w2.shape[1], w3.shape[1]
    chans = (c0, c1, c2, c3)
    n_nodes = B * l
    eps = 1e-5

    g_sz = 256 // c0 if (256 % c0 == 0 and all(c == c0 for c in chans)) else 1

    bt = B
    for cand in (64, 32, 16, 8, 4, 2, 1):
        if B % cand == 0 and cand % g_sz == 0:
            bt = cand
            break
    nt = B // bt
    groups = bt // g_sz

    s = _normalized_adjacency(edge_index, l)

    eye = jnp.eye(g_sz, dtype=jnp.float32)
    wk = (jnp.kron(eye, w1), jnp.kron(eye, w2), jnp.kron(eye, w3))

    x2 = x.reshape(B * c0, l)

    cp = pltpu.CompilerParams(dimension_semantics=("parallel",),
                              vmem_limit_bytes=48 * 1024 * 1024)

    def act_spec(cw):
        return pl.BlockSpec((l, bt * cw), lambda j: (0, j))

    def full_spec(shape):
        nd = len(shape)
        return pl.BlockSpec(tuple(shape), lambda j: (0,) * nd)

    def stats_spec(cw):
        return pl.BlockSpec((1, 1, bt * cw), lambda j: (j, 0, 0))

    def stats_shape(cw):
        return jax.ShapeDtypeStruct((nt, 1, bt * cw), jnp.float32)

    def p_fold(cw):
        return jnp.tile(jnp.eye(cw, dtype=jnp.float32), (bt, 1))

    act_dtype = jnp.bfloat16

    agg1, ps1, pq1 = pl.pallas_call(
        functools.partial(_layer1_kernel, groups=groups, gin=g_sz * c0),
        grid=(nt,),
        in_specs=[pl.BlockSpec((bt * c0, l), lambda j: (j, 0)),
                  full_spec(wk[0].shape), full_spec(s.shape)],
        out_specs=(act_spec(c1), stats_spec(c1), stats_spec(c1)),
        out_shape=(jax.ShapeDtypeStruct((l, B * c1), act_dtype),
                   stats_shape(c1), stats_shape(c1)),
        compiler_params=cp,
    )(x2, wk[0], s)

    def run_layer(h, ps, pq, gamma, beta, w_blk, cin, cout):
        pf = p_fold(cin)
        g2d, b2d = gamma.reshape(1, cin), beta.reshape(1, cin)
        return pl.pallas_call(
            functools.partial(_layer_kernel, groups=groups, gin=g_sz * cin,
                              n_nodes=float(n_nodes), eps=eps),
            grid=(nt,),
            in_specs=[act_spec(cin), full_spec(ps.shape), full_spec(pq.shape),
                      full_spec(g2d.shape), full_spec(b2d.shape),
                      full_spec(pf.shape), full_spec(w_blk.shape),
                      full_spec(s.shape)],
            out_specs=(act_spec(cout), stats_spec(cout), stats_spec(cout)),
            out_shape=(jax.ShapeDtypeStruct((l, B * cout), act_dtype),
                       stats_shape(cout), stats_shape(cout)),
            compiler_params=cp,
        )(h, ps, pq, g2d, b2d, pf, w_blk, s)

    agg2, ps2, pq2 = run_layer(agg1, ps1, pq1, g1, be1, wk[1], c1, c2)
    agg3, ps3, pq3 = run_layer(agg2, ps2, pq2, g2, be2, wk[2], c2, c3)

    eye_l = jnp.eye(l, dtype=jnp.float32)
    pf3 = p_fold(c3)
    g3d, be3d = g3.reshape(1, c3), be3.reshape(1, c3)
    y = pl.pallas_call(
        functools.partial(_bn_relu_t_kernel, groups=groups, gout=g_sz * c3,
                          bt=bt, cout=c3, n_nodes=float(n_nodes), eps=eps),
        grid=(nt,),
        in_specs=[act_spec(c3), full_spec(ps3.shape), full_spec(pq3.shape),
                  full_spec(g3d.shape), full_spec(be3d.shape),
                  full_spec(pf3.shape), full_spec(eye_l.shape)],
        out_specs=pl.BlockSpec((bt, c3, l), lambda j: (j, 0, 0)),
        out_shape=jax.ShapeDtypeStruct((B, c3, l), jnp.float32),
        compiler_params=cp,
    )(agg3, ps3, pq3, g3d, be3d, pf3, eye_l)

    return y


# speedup vs baseline: 1.4295x; 1.4295x over previous
import functools
import math

import jax
import jax.numpy as jnp
from jax.experimental import pallas as pl
from jax.experimental.pallas import tpu as pltpu


def _stats(agg, sum_ref, sq_ref):
    sum_ref[...] = jnp.sum(agg, axis=0, keepdims=True)[None]
    sq_ref[...] = jnp.sum(agg * agg, axis=0, keepdims=True)[None]


def _layer1_kernel(x_ref, w_ref, s_ref, agg_ref, sum_ref, sq_ref, *, groups, gin):
    parts = []
    for i in range(groups):
        xg = x_ref[pl.ds(i * gin, gin), :]
        parts.append(jax.lax.dot_general(
            xg, w_ref[...], (((0,), (0,)), ((), ())),
            preferred_element_type=jnp.float32))
    hw = jnp.concatenate(parts, axis=1)
    agg = jnp.dot(s_ref[...], hw, preferred_element_type=jnp.float32)
    agg_ref[...] = agg.astype(agg_ref.dtype)
    _stats(agg, sum_ref, sq_ref)


def _layer_kernel(h_ref, scale_ref, shift_ref, w_ref, s_ref,
                  agg_ref, sum_ref, sq_ref, *, groups, gin):
    h = jnp.maximum(h_ref[...].astype(jnp.float32) * scale_ref[...]
                    + shift_ref[...], 0.0)
    parts = []
    for i in range(groups):
        hg = h[:, i * gin:(i + 1) * gin]
        parts.append(jnp.dot(hg, w_ref[...],
                             preferred_element_type=jnp.float32))
    hw = jnp.concatenate(parts, axis=1)
    agg = jnp.dot(s_ref[...], hw, preferred_element_type=jnp.float32)
    agg_ref[...] = agg.astype(agg_ref.dtype)
    _stats(agg, sum_ref, sq_ref)


def _bn_relu_t_kernel(agg_ref, scale_ref, shift_ref, eye_ref, o_ref, *,
                      groups, gout, bt, cout):
    y = jnp.maximum(agg_ref[...].astype(jnp.float32) * scale_ref[...]
                    + shift_ref[...], 0.0)
    parts = []
    for i in range(groups):
        yg = y[:, i * gout:(i + 1) * gout]
        parts.append(jax.lax.dot_general(
            yg, eye_ref[...], (((0,), (0,)), ((), ())),
            preferred_element_type=jnp.float32))
    o_ref[...] = jnp.concatenate(parts, axis=0).reshape(bt, cout, -1)


def _normalized_adjacency(edge_index, num_nodes):
    src, dst = edge_index[0], edge_index[1]
    oh_src = jax.nn.one_hot(src, num_nodes, dtype=jnp.float32)
    oh_dst = jax.nn.one_hot(dst, num_nodes, dtype=jnp.float32)
    a = jax.lax.dot_general(oh_dst, oh_src, (((0,), (0,)), ((), ())))
    a = a + 2.0 * jnp.eye(num_nodes, dtype=jnp.float32)
    deg = jnp.sum(a, axis=1)
    dinv = jnp.where(deg > 0, jax.lax.rsqrt(deg), 0.0)
    return dinv[:, None] * a * dinv[None, :]


def kernel(x, edge_index, w1, b1, g1, be1, w2, b2, g2, be2, w3, b3, g3, be3):
    b, n, c0, l = x.shape
    B = b * n
    c1, c2, c3 = w1.shape[1], w2.shape[1], w3.shape[1]
    chans = (c0, c1, c2, c3)
    n_nodes = B * l
    eps = 1e-5

    g_sz = 256 // c0 if (256 % c0 == 0 and all(c == c0 for c in chans)) else 1

    bt = B
    for cand in (128, 64, 32, 16, 8, 4, 2, 1):
        if B % cand == 0 and cand % g_sz == 0:
            bt = cand
            break
    nt = B // bt
    groups = bt // g_sz

    s = _normalized_adjacency(edge_index, l)

    eye = jnp.eye(g_sz, dtype=jnp.float32)
    wk = (jnp.kron(eye, w1), jnp.kron(eye, w2), jnp.kron(eye, w3))

    x2 = x.reshape(B * c0, l)

    cp = pltpu.CompilerParams(dimension_semantics=("parallel",),
                              vmem_limit_bytes=48 * 1024 * 1024)

    def act_spec(cw):
        return pl.BlockSpec((l, bt * cw), lambda j: (0, j))

    def full_spec(shape):
        nd = len(shape)
        return pl.BlockSpec(tuple(shape), lambda j: (0,) * nd)

    def stats_spec(cw):
        return pl.BlockSpec((1, 1, bt * cw), lambda j: (j, 0, 0))

    def stats_shape(cw):
        return jax.ShapeDtypeStruct((nt, 1, bt * cw), jnp.float32)

    def fold_stats(psum, psq, gamma, beta, cout):
        tot = psum.reshape(-1, cout).sum(axis=0)
        tot2 = psq.reshape(-1, cout).sum(axis=0)
        mean = tot / n_nodes
        var = tot2 / n_nodes - mean * mean
        scale = gamma * jax.lax.rsqrt(var + eps)
        shift = beta - mean * scale
        return (jnp.tile(scale, bt).reshape(1, bt * cout),
                jnp.tile(shift, bt).reshape(1, bt * cout))

    act_dtype = jnp.bfloat16

    agg1, ps1, pq1 = pl.pallas_call(
        functools.partial(_layer1_kernel, groups=groups, gin=g_sz * c0),
        grid=(nt,),
        in_specs=[pl.BlockSpec((bt * c0, l), lambda j: (j, 0)),
                  full_spec(wk[0].shape), full_spec(s.shape)],
        out_specs=(act_spec(c1), stats_spec(c1), stats_spec(c1)),
        out_shape=(jax.ShapeDtypeStruct((l, B * c1), act_dtype),
                   stats_shape(c1), stats_shape(c1)),
        compiler_params=cp,
    )(x2, wk[0], s)
    sc1, sh1 = fold_stats(ps1, pq1, g1, be1, c1)

    def run_layer(h, w_blk, cin, cout, scale, shift):
        return pl.pallas_call(
            functools.partial(_layer_kernel, groups=groups, gin=g_sz * cin),
            grid=(nt,),
            in_specs=[act_spec(cin), full_spec(scale.shape),
                      full_spec(shift.shape), full_spec(w_blk.shape),
                      full_spec(s.shape)],
            out_specs=(act_spec(cout), stats_spec(cout), stats_spec(cout)),
            out_shape=(jax.ShapeDtypeStruct((l, B * cout), act_dtype),
                       stats_shape(cout), stats_shape(cout)),
            compiler_params=cp,
        )(h, scale, shift, w_blk, s)

    agg2, ps2, pq2 = run_layer(agg1, wk[1], c1, c2, sc1, sh1)
    sc2, sh2 = fold_stats(ps2, pq2, g2, be2, c2)
    agg3, ps3, pq3 = run_layer(agg2, wk[2], c2, c3, sc2, sh2)
    sc3, sh3 = fold_stats(ps3, pq3, g3, be3, c3)

    eye_l = jnp.eye(l, dtype=jnp.float32)
    y = pl.pallas_call(
        functools.partial(_bn_relu_t_kernel, groups=groups, gout=g_sz * c3,
                          bt=bt, cout=c3),
        grid=(nt,),
        in_specs=[act_spec(c3), full_spec(sc3.shape), full_spec(sh3.shape),
                  full_spec(eye_l.shape)],
        out_specs=pl.BlockSpec((bt, c3, l), lambda j: (j, 0, 0)),
        out_shape=jax.ShapeDtypeStruct((B, c3, l), jnp.float32),
        compiler_params=cp,
    )(agg3, sc3, sh3, eye_l)

    return y


# speedup vs baseline: 1.4915x; 1.0433x over previous
import functools
import math

import jax
import jax.numpy as jnp
from jax.experimental import pallas as pl
from jax.experimental.pallas import tpu as pltpu


def _stats(agg, sum_ref, sq_ref):
    sum_ref[...] = jnp.sum(agg, axis=0, keepdims=True)[None]
    sq_ref[...] = jnp.sum(agg * agg, axis=0, keepdims=True)[None]


def _layer1_kernel(x_ref, w_ref, s_ref, agg_ref, sum_ref, sq_ref, *, groups, gin):
    parts = []
    for i in range(groups):
        xg = x_ref[pl.ds(i * gin, gin), :]
        parts.append(jax.lax.dot_general(
            xg, w_ref[...], (((0,), (0,)), ((), ())),
            preferred_element_type=jnp.float32))
    hw = jnp.concatenate(parts, axis=1)
    agg = jnp.dot(s_ref[...], hw, preferred_element_type=jnp.float32)
    agg_ref[...] = agg.astype(agg_ref.dtype)
    _stats(agg, sum_ref, sq_ref)


def _layer_kernel(h_ref, scale_ref, shift_ref, w_ref, s_ref,
                  agg_ref, sum_ref, sq_ref, *, groups, gin):
    h = jnp.maximum(h_ref[...].astype(jnp.float32) * scale_ref[...]
                    + shift_ref[...], 0.0)
    parts = []
    for i in range(groups):
        hg = h[:, i * gin:(i + 1) * gin]
        parts.append(jnp.dot(hg, w_ref[...],
                             preferred_element_type=jnp.float32))
    hw = jnp.concatenate(parts, axis=1)
    agg = jnp.dot(s_ref[...], hw, preferred_element_type=jnp.float32)
    agg_ref[...] = agg.astype(agg_ref.dtype)
    _stats(agg, sum_ref, sq_ref)


def _bn_relu_t_kernel(agg_ref, scale_ref, shift_ref, eye_ref, o_ref, *,
                      groups, gout, bt, cout):
    y = jnp.maximum(agg_ref[...].astype(jnp.float32) * scale_ref[...]
                    + shift_ref[...], 0.0)
    parts = []
    for i in range(groups):
        yg = y[:, i * gout:(i + 1) * gout]
        parts.append(jax.lax.dot_general(
            yg, eye_ref[...], (((0,), (0,)), ((), ())),
            preferred_element_type=jnp.float32))
    o_ref[...] = jnp.concatenate(parts, axis=0).reshape(bt, cout, -1)


def _normalized_adjacency(edge_index, num_nodes):
    src, dst = edge_index[0], edge_index[1]
    oh_src = jax.nn.one_hot(src, num_nodes, dtype=jnp.float32)
    oh_dst = jax.nn.one_hot(dst, num_nodes, dtype=jnp.float32)
    a = jax.lax.dot_general(oh_dst, oh_src, (((0,), (0,)), ((), ())))
    a = a + 2.0 * jnp.eye(num_nodes, dtype=jnp.float32)
    deg = jnp.sum(a, axis=1)
    dinv = jnp.where(deg > 0, jax.lax.rsqrt(deg), 0.0)
    return dinv[:, None] * a * dinv[None, :]


def kernel(x, edge_index, w1, b1, g1, be1, w2, b2, g2, be2, w3, b3, g3, be3):
    b, n, c0, l = x.shape
    B = b * n
    c1, c2, c3 = w1.shape[1], w2.shape[1], w3.shape[1]
    chans = (c0, c1, c2, c3)
    n_nodes = B * l
    eps = 1e-5

    g_sz = 256 // c0 if (256 % c0 == 0 and all(c == c0 for c in chans)) else 1

    bt = B
    for cand in (256, 128, 64, 32, 16, 8, 4, 2, 1):
        if B % cand == 0 and cand % g_sz == 0:
            bt = cand
            break
    nt = B // bt
    groups = bt // g_sz

    s = _normalized_adjacency(edge_index, l)

    eye = jnp.eye(g_sz, dtype=jnp.float32)
    wk = (jnp.kron(eye, w1), jnp.kron(eye, w2), jnp.kron(eye, w3))

    x2 = x.reshape(B * c0, l)

    cp = pltpu.CompilerParams(dimension_semantics=("parallel",),
                              vmem_limit_bytes=48 * 1024 * 1024)

    def act_spec(cw):
        return pl.BlockSpec((l, bt * cw), lambda j: (0, j))

    def full_spec(shape):
        nd = len(shape)
        return pl.BlockSpec(tuple(shape), lambda j: (0,) * nd)

    def stats_spec(cw):
        return pl.BlockSpec((1, 1, bt * cw), lambda j: (j, 0, 0))

    def stats_shape(cw):
        return jax.ShapeDtypeStruct((nt, 1, bt * cw), jnp.float32)

    def fold_stats(psum, psq, gamma, beta, cout):
        tot = psum.reshape(-1, cout).sum(axis=0)
        tot2 = psq.reshape(-1, cout).sum(axis=0)
        mean = tot / n_nodes
        var = tot2 / n_nodes - mean * mean
        scale = gamma * jax.lax.rsqrt(var + eps)
        shift = beta - mean * scale
        return (jnp.tile(scale, bt).reshape(1, bt * cout),
                jnp.tile(shift, bt).reshape(1, bt * cout))

    act_dtype = jnp.bfloat16

    agg1, ps1, pq1 = pl.pallas_call(
        functools.partial(_layer1_kernel, groups=groups, gin=g_sz * c0),
        grid=(nt,),
        in_specs=[pl.BlockSpec((bt * c0, l), lambda j: (j, 0)),
                  full_spec(wk[0].shape), full_spec(s.shape)],
        out_specs=(act_spec(c1), stats_spec(c1), stats_spec(c1)),
        out_shape=(jax.ShapeDtypeStruct((l, B * c1), act_dtype),
                   stats_shape(c1), stats_shape(c1)),
        compiler_params=cp,
    )(x2, wk[0], s)
    sc1, sh1 = fold_stats(ps1, pq1, g1, be1, c1)

    def run_layer(h, w_blk, cin, cout, scale, shift):
        return pl.pallas_call(
            functools.partial(_layer_kernel, groups=groups, gin=g_sz * cin),
            grid=(nt,),
            in_specs=[act_spec(cin), full_spec(scale.shape),
                      full_spec(shift.shape), full_spec(w_blk.shape),
                      full_spec(s.shape)],
            out_specs=(act_spec(cout), stats_spec(cout), stats_spec(cout)),
            out_shape=(jax.ShapeDtypeStruct((l, B * cout), act_dtype),
                       stats_shape(cout), stats_shape(cout)),
            compiler_params=cp,
        )(h, scale, shift, w_blk, s)

    agg2, ps2, pq2 = run_layer(agg1, wk[1], c1, c2, sc1, sh1)
    sc2, sh2 = fold_stats(ps2, pq2, g2, be2, c2)
    agg3, ps3, pq3 = run_layer(agg2, wk[2], c2, c3, sc2, sh2)
    sc3, sh3 = fold_stats(ps3, pq3, g3, be3, c3)

    eye_l = jnp.eye(l, dtype=jnp.float32)
    y = pl.pallas_call(
        functools.partial(_bn_relu_t_kernel, groups=groups, gout=g_sz * c3,
                          bt=bt, cout=c3),
        grid=(nt,),
        in_specs=[act_spec(c3), full_spec(sc3.shape), full_spec(sh3.shape),
                  full_spec(eye_l.shape)],
        out_specs=pl.BlockSpec((bt, c3, l), lambda j: (j, 0, 0)),
        out_shape=jax.ShapeDtypeStruct((B, c3, l), jnp.float32),
        compiler_params=cp,
    )(agg3, sc3, sh3, eye_l)

    return y


# speedup vs baseline: 1.5135x; 1.0148x over previous
import functools
import math

import jax
import jax.numpy as jnp
from jax.experimental import pallas as pl
from jax.experimental.pallas import tpu as pltpu


def _stats(agg, sum_ref, sq_ref):
    sum_ref[...] = jnp.sum(agg, axis=0, keepdims=True)[None]
    sq_ref[...] = jnp.sum(agg * agg, axis=0, keepdims=True)[None]


def _adjacency(ei_ref, l):
    e2 = ei_ref.shape[-1]
    src = ei_ref[0:1, :].astype(jnp.int32)
    dst = ei_ref[1:2, :].astype(jnp.int32)
    rows = jax.lax.broadcasted_iota(jnp.int32, (l, e2), 0)
    ohs = (rows == src).astype(jnp.float32)
    ohd = (rows == dst).astype(jnp.float32)
    dn = (((1,), (1,)), ((), ()))
    a = jax.lax.dot_general(ohd, ohs, dn,
                            preferred_element_type=jnp.float32)
    at = jax.lax.dot_general(ohs, ohd, dn,
                             preferred_element_type=jnp.float32)
    ii = jax.lax.broadcasted_iota(jnp.int32, (l, l), 0)
    jj = jax.lax.broadcasted_iota(jnp.int32, (l, l), 1)
    eye2 = jnp.where(ii == jj, 2.0, 0.0).astype(jnp.float32)
    a = a + eye2
    deg_col = jnp.sum(a, axis=1, keepdims=True)
    deg_row = jnp.sum(at + eye2, axis=0, keepdims=True)
    dinv_col = jnp.where(deg_col > 0, jax.lax.rsqrt(deg_col), 0.0)
    dinv_row = jnp.where(deg_row > 0, jax.lax.rsqrt(deg_row), 0.0)
    return dinv_col * a * dinv_row


def _block_diag(w_ref, g_sz):
    cin, cout = w_ref.shape
    wt = jnp.concatenate([w_ref[...]] * g_sz, axis=0)
    wt = jnp.concatenate([wt] * g_sz, axis=1)
    bi = jax.lax.broadcasted_iota(jnp.int32, wt.shape, 0) // cin
    bj = jax.lax.broadcasted_iota(jnp.int32, wt.shape, 1) // cout
    return jnp.where(bi == bj, wt, 0.0)


def _layer1_kernel(x_ref, ei_ref, w_ref, agg_ref, sum_ref, sq_ref, s_ref,
                   *, groups, gin, g_sz, l):
    s = _adjacency(ei_ref, l)
    s_ref[...] = s
    wk = _block_diag(w_ref, g_sz)
    parts = []
    for i in range(groups):
        xg = x_ref[pl.ds(i * gin, gin), :]
        parts.append(jax.lax.dot_general(
            xg, wk, (((0,), (0,)), ((), ())),
            preferred_element_type=jnp.float32))
    hw = jnp.concatenate(parts, axis=1)
    agg = jnp.dot(s, hw, preferred_element_type=jnp.float32)
    agg_ref[...] = agg.astype(agg_ref.dtype)
    _stats(agg, sum_ref, sq_ref)


def _layer_kernel(h_ref, scale_ref, shift_ref, w_ref, s_ref,
                  agg_ref, sum_ref, sq_ref, *, groups, gin, g_sz):
    wk = _block_diag(w_ref, g_sz)
    h = jnp.maximum(h_ref[...].astype(jnp.float32) * scale_ref[...]
                    + shift_ref[...], 0.0)
    parts = []
    for i in range(groups):
        hg = h[:, i * gin:(i + 1) * gin]
        parts.append(jnp.dot(hg, wk, preferred_element_type=jnp.float32))
    hw = jnp.concatenate(parts, axis=1)
    agg = jnp.dot(s_ref[...], hw, preferred_element_type=jnp.float32)
    agg_ref[...] = agg.astype(agg_ref.dtype)
    _stats(agg, sum_ref, sq_ref)


def _bn_relu_t_kernel(agg_ref, scale_ref, shift_ref, o_ref, *,
                      groups, gout, bt, cout, l):
    ii = jax.lax.broadcasted_iota(jnp.int32, (l, l), 0)
    jj = jax.lax.broadcasted_iota(jnp.int32, (l, l), 1)
    eye_l = jnp.where(ii == jj, 1.0, 0.0).astype(jnp.float32)
    y = jnp.maximum(agg_ref[...].astype(jnp.float32) * scale_ref[...]
                    + shift_ref[...], 0.0)
    parts = []
    for i in range(groups):
        yg = y[:, i * gout:(i + 1) * gout]
        parts.append(jax.lax.dot_general(
            yg, eye_l, (((0,), (0,)), ((), ())),
            preferred_element_type=jnp.float32))
    o_ref[...] = jnp.concatenate(parts, axis=0).reshape(bt, cout, -1)


def kernel(x, edge_index, w1, b1, g1, be1, w2, b2, g2, be2, w3, b3, g3, be3):
    b, n, c0, l = x.shape
    B = b * n
    c1, c2, c3 = w1.shape[1], w2.shape[1], w3.shape[1]
    chans = (c0, c1, c2, c3)
    n_nodes = B * l
    eps = 1e-5

    g_sz = 256 // c0 if (256 % c0 == 0 and all(c == c0 for c in chans)) else 1

    bt = B
    for cand in (256, 128, 64, 32, 16, 8, 4, 2, 1):
        if B % cand == 0 and cand % g_sz == 0:
            bt = cand
            break
    nt = B // bt
    groups = bt // g_sz

    x2 = x.reshape(B * c0, l)

    cp = pltpu.CompilerParams(dimension_semantics=("parallel",),
                              vmem_limit_bytes=48 * 1024 * 1024)

    def act_spec(cw):
        return pl.BlockSpec((l, bt * cw), lambda j: (0, j))

    def full_spec(shape):
        nd = len(shape)
        return pl.BlockSpec(tuple(shape), lambda j: (0,) * nd)

    def stats_spec(cw):
        return pl.BlockSpec((1, 1, bt * cw), lambda j: (j, 0, 0))

    def stats_shape(cw):
        return jax.ShapeDtypeStruct((nt, 1, bt * cw), jnp.float32)

    def fold_stats(psum, psq, gamma, beta, cout):
        tot = psum.reshape(-1, cout).sum(axis=0)
        tot2 = psq.reshape(-1, cout).sum(axis=0)
        mean = tot / n_nodes
        var = tot2 / n_nodes - mean * mean
        scale = gamma * jax.lax.rsqrt(var + eps)
        shift = beta - mean * scale
        return (jnp.tile(scale, bt).reshape(1, bt * cout),
                jnp.tile(shift, bt).reshape(1, bt * cout))

    act_dtype = jnp.bfloat16

    agg1, ps1, pq1, s_arr = pl.pallas_call(
        functools.partial(_layer1_kernel, groups=groups, gin=g_sz * c0,
                          g_sz=g_sz, l=l),
        grid=(nt,),
        in_specs=[pl.BlockSpec((bt * c0, l), lambda j: (j, 0)),
                  full_spec(edge_index.shape), full_spec(w1.shape)],
        out_specs=(act_spec(c1), stats_spec(c1), stats_spec(c1),
                   full_spec((l, l))),
        out_shape=(jax.ShapeDtypeStruct((l, B * c1), act_dtype),
                   stats_shape(c1), stats_shape(c1),
                   jax.ShapeDtypeStruct((l, l), jnp.float32)),
        compiler_params=cp,
    )(x2, edge_index, w1)
    sc1, sh1 = fold_stats(ps1, pq1, g1, be1, c1)

    def run_layer(h, w_raw, cin, cout, scale, shift):
        return pl.pallas_call(
            functools.partial(_layer_kernel, groups=groups, gin=g_sz * cin,
                              g_sz=g_sz),
            grid=(nt,),
            in_specs=[act_spec(cin), full_spec(scale.shape),
                      full_spec(shift.shape), full_spec(w_raw.shape),
                      full_spec((l, l))],
            out_specs=(act_spec(cout), stats_spec(cout), stats_spec(cout)),
            out_shape=(jax.ShapeDtypeStruct((l, B * cout), act_dtype),
                       stats_shape(cout), stats_shape(cout)),
            compiler_params=cp,
        )(h, scale, shift, w_raw, s_arr)

    agg2, ps2, pq2 = run_layer(agg1, w2, c1, c2, sc1, sh1)
    sc2, sh2 = fold_stats(ps2, pq2, g2, be2, c2)
    agg3, ps3, pq3 = run_layer(agg2, w3, c2, c3, sc2, sh2)
    sc3, sh3 = fold_stats(ps3, pq3, g3, be3, c3)

    y = pl.pallas_call(
        functools.partial(_bn_relu_t_kernel, groups=groups, gout=g_sz * c3,
                          bt=bt, cout=c3, l=l),
        grid=(nt,),
        in_specs=[act_spec(c3), full_spec(sc3.shape), full_spec(sh3.shape)],
        out_specs=pl.BlockSpec((bt, c3, l), lambda j: (j, 0, 0)),
        out_shape=jax.ShapeDtypeStruct((B, c3, l), jnp.float32),
        compiler_params=cp,
    )(agg3, sc3, sh3)

    return y
